# baseline (speedup 1.0000x reference)
import functools

import jax
import jax.numpy as jnp
from jax import lax
from jax.experimental import pallas as pl
from jax.experimental.pallas import tpu as pltpu
from jax.experimental.pallas import tpu_sc as plsc

B = 16384
FACTOR = 32
MLP_DIM = 64
UN = 1000000
IT = 100000
NC = 2
NS = 16
NW = NC * NS
BPW = B // NW
CMAXU = (UN - 1) // 128
CMAXI = (IT - 1) // 128
TAILU = UN - CMAXU * 128
TAILI = IT - CMAXI * 128
NB = 2


def _phase(t2m, t2g, tlm_hbm, tlg_hbm, sidx_hbm, perm_hbm, out_flat,
           sj, pj, runs, vtmp, vtmp2, bufm, bufg, stage, tailm, tailg,
           semm, semg, semw, *, cmax, tail, base):
    pltpu.sync_copy(tlm_hbm, tailm.at[pl.ds(0, tail)])
    pltpu.sync_copy(tlg_hbm, tailg.at[pl.ds(0, tail)])

    pltpu.sync_copy(sidx_hbm.at[pl.ds(base, BPW)], vtmp)
    pltpu.sync_copy(perm_hbm.at[pl.ds(base, BPW)], vtmp2)

    def smem_fill(k, carry):
        vs = vtmp[pl.ds(k * 16, 16)]
        vp = vtmp2[pl.ds(k * 16, 16)]
        for t in range(16):
            sj[k * 16 + t] = vs[t]
            pj[k * 16 + t] = vp[t]
        return carry

    lax.fori_loop(0, BPW // 16, smem_fill, 0)

    def col_of(k):
        return lax.shift_right_logical(sj[k], 7)

    runs[0] = 0

    def scan_body(j, st):
        r, c = st
        cj = col_of(j)
        is_new = cj != c

        @pl.when(is_new)
        def _():
            runs[r] = j

        return r + jnp.where(is_new, 1, 0), cj

    nrun, _ = lax.fori_loop(1, BPW, scan_body, (jnp.int32(1), col_of(0)))
    runs[nrun] = BPW

    def dmas(c, p):
        lanes = pl.ds(c * 128, 128)
        return (
            pltpu.make_async_copy(
                t2m.at[:, lanes], bufm.at[pl.ds(p * MLP_DIM, MLP_DIM)], semm),
            pltpu.make_async_copy(
                t2g.at[:, lanes], bufg.at[pl.ds(p * FACTOR, FACTOR)], semg),
        )

    def fire(r):
        c = col_of(runs[r])

        @pl.when(c != cmax)
        def _():
            for cp in dmas(c, lax.rem(r, NB)):
                cp.start()

    def drain(c, p):
        @pl.when(c != cmax)
        def _():
            for cp in dmas(c, p):
                cp.wait()

    iota16 = lax.iota(jnp.int32, 16)

    def extract(j, p):
        lvec = jnp.full((16,), lax.rem(sj[j], 128), jnp.int32)
        pvec = jnp.full((16,), p, jnp.int32)
        c = col_of(j)

        @pl.when(c != cmax)
        def _():
            for q in range(MLP_DIM // 16):
                f = iota16 + (16 * q)
                row = plsc.load_gather(bufm, [pvec * MLP_DIM + f, lvec])
                stage[j, pl.ds(16 * q, 16)] = row
            for q in range(FACTOR // 16):
                f = iota16 + (16 * q)
                row = plsc.load_gather(bufg, [pvec * FACTOR + f, lvec])
                stage[j, pl.ds(MLP_DIM + 16 * q, 16)] = row

        @pl.when(c == cmax)
        def _():
            for q in range(MLP_DIM // 16):
                f = iota16 + (16 * q)
                row = plsc.load_gather(tailm, [lvec, f])
                stage[j, pl.ds(16 * q, 16)] = row
            for q in range(FACTOR // 16):
                f = iota16 + (16 * q)
                row = plsc.load_gather(tailg, [lvec, f])
                stage[j, pl.ds(MLP_DIM + 16 * q, 16)] = row

    lax.fori_loop(0, jnp.minimum(NB - 1, nrun), lambda r, c: (fire(r), c)[1], 0)

    def run_body(r, carry):
        p = lax.rem(r, NB)
        drain(col_of(runs[r]), p)

        @pl.when(r + NB - 1 < nrun)
        def _():
            fire(r + NB - 1)

        def ext(k, c2):
            extract(k, p)
            return c2

        lax.fori_loop(runs[r], runs[r + 1], ext, 0)
        return carry

    lax.fori_loop(0, nrun, run_body, 0)

    def flush(k, carry):
        handles = []
        for t in range(16):
            j = k * 16 + t
            cp = pltpu.make_async_copy(
                stage.at[j], out_flat.at[pl.ds(pj[j] * 128, 128)], semw)
            cp.start()
            handles.append(cp)
        for cp in handles:
            cp.wait()
        return carry

    lax.fori_loop(0, BPW // 16, flush, 0)


def _sc_body(su, pu, si, pi, t2um, t2ug, t2im, t2ig, tlum, tlug, tlim, tlig,
             o_user, o_item, sj, pj, runs, vtmp, vtmp2, bufm, bufg, stage,
             tailm, tailg, semm, semg, semw):
    wid = lax.axis_index("s") * NC + lax.axis_index("c")
    base = wid * BPW
    _phase(t2um, t2ug, tlum, tlug, su, pu, o_user,
           sj, pj, runs, vtmp, vtmp2, bufm, bufg, stage, tailm, tailg,
           semm, semg, semw, cmax=CMAXU, tail=TAILU, base=base)
    _phase(t2im, t2ig, tlim, tlig, si, pi, o_item,
           sj, pj, runs, vtmp, vtmp2, bufm, bufg, stage, tailm, tailg,
           semm, semg, semw, cmax=CMAXI, tail=TAILI, base=base)


@functools.cache
def _sc_gather():
    return pl.kernel(
        _sc_body,
        out_type=[
            jax.ShapeDtypeStruct((B * 128,), jnp.float32),
            jax.ShapeDtypeStruct((B * 128,), jnp.float32),
        ],
        mesh=plsc.VectorSubcoreMesh(core_axis_name="c", subcore_axis_name="s",
                                    num_cores=NC, num_subcores=NS),
        compiler_params=pltpu.CompilerParams(needs_layout_passes=False),
        scratch_types=[
            pltpu.SMEM((BPW,), jnp.int32),
            pltpu.SMEM((BPW,), jnp.int32),
            pltpu.SMEM((BPW + 2,), jnp.int32),
            pltpu.VMEM((BPW,), jnp.int32),
            pltpu.VMEM((BPW,), jnp.int32),
            pltpu.VMEM((NB * MLP_DIM, 128), jnp.float32),
            pltpu.VMEM((NB * FACTOR, 128), jnp.float32),
            pltpu.VMEM((BPW, 128), jnp.float32),
            pltpu.VMEM((TAILU, MLP_DIM), jnp.float32),
            pltpu.VMEM((TAILU, FACTOR), jnp.float32),
            pltpu.SemaphoreType.DMA,
            pltpu.SemaphoreType.DMA,
            pltpu.SemaphoreType.DMA,
        ],
    )


BLK = 2048


def _dense_body(u_ref, i_ref, w0a_ref, w0b_ref, b0_ref,
                w1_ref, b1_ref, wpa_ref, wpb_ref, bp_ref, out_ref):
    ub = u_ref[...]
    ib = i_ref[...]
    um = ub[:, :MLP_DIM]
    ug = ub[:, MLP_DIM:MLP_DIM + FACTOR]
    im = ib[:, :MLP_DIM]
    ig = ib[:, MLP_DIM:MLP_DIM + FACTOR]
    h = jnp.dot(um, w0a_ref[...], preferred_element_type=jnp.float32)
    h += jnp.dot(im, w0b_ref[...], preferred_element_type=jnp.float32)
    h = jnp.maximum(h + b0_ref[...], 0.0)
    m = jnp.maximum(
        jnp.dot(h, w1_ref[...], preferred_element_type=jnp.float32)
        + b1_ref[...], 0.0)
    g = ug * ig
    out_ref[...] = (jnp.sum(m * wpb_ref[...], axis=1)
                    + jnp.sum(g * wpa_ref[...], axis=1) + bp_ref[...][0, 0])


def _dense(u, i, w0a, w0b, b0, w1, b1, wpa, wpb, bp):
    full = lambda r, c_: pl.BlockSpec((r, c_), lambda k: (0, 0))
    return pl.pallas_call(
        _dense_body,
        grid=(B // BLK,),
        in_specs=[
            pl.BlockSpec((BLK, 128), lambda k: (k, 0)),
            pl.BlockSpec((BLK, 128), lambda k: (k, 0)),
            full(MLP_DIM, MLP_DIM),
            full(MLP_DIM, MLP_DIM),
            full(1, MLP_DIM),
            full(MLP_DIM, FACTOR),
            full(1, FACTOR),
            full(1, FACTOR),
            full(1, FACTOR),
            full(1, 1),
        ],
        out_specs=pl.BlockSpec((BLK,), lambda k: (k,)),
        out_shape=jax.ShapeDtypeStruct((B,), jnp.float32),
    )(u, i, w0a, w0b, b0, w1, b1, wpa, wpb, bp)


def kernel(user, item, emb_user_gmf, emb_item_gmf, emb_user_mlp, emb_item_mlp,
           W0, b0, W1, b1, Wp, bp):
    pos = lax.iota(jnp.int32, B)
    su, pu = lax.sort_key_val(user, pos)
    si, pi = lax.sort_key_val(item, pos)
    ou, oi = _sc_gather()(
        su, pu, si, pi,
        emb_user_mlp.T, emb_user_gmf.T, emb_item_mlp.T, emb_item_gmf.T,
        emb_user_mlp[UN - TAILU:], emb_user_gmf[UN - TAILU:],
        emb_item_mlp[IT - TAILI:], emb_item_gmf[IT - TAILI:])
    u2 = ou.reshape(B, 128)
    i2 = oi.reshape(B, 128)
    return _dense(u2, i2, W0[:MLP_DIM], W0[MLP_DIM:], b0.reshape(1, MLP_DIM),
                  W1, b1.reshape(1, FACTOR), Wp[:FACTOR].reshape(1, FACTOR),
                  Wp[FACTOR:].reshape(1, FACTOR), bp.reshape(1, 1))


# speedup vs baseline: 1.3801x; 1.3801x over previous
import functools

import jax
import jax.numpy as jnp
from jax import lax
from jax.experimental import pallas as pl
from jax.experimental.pallas import tpu as pltpu
from jax.experimental.pallas import tpu_sc as plsc

B = 16384
FACTOR = 32
MLP_DIM = 64
UN = 1000000
IT = 100000
NC = 2
NS = 16
NW = NC * NS
BPW = B // NW
CMAXU = (UN - 1) // 128
CMAXI = (IT - 1) // 128
TAILU = UN - CMAXU * 128
TAILI = IT - CMAXI * 128
NB = 4


def _phase(t2m, t2g, tlm_hbm, tlg_hbm, sidx_hbm, perm_hbm, out_flat,
           sj, pj, runs, vtmp, vtmp2, bufm, bufg, stage, tailm, tailg,
           semm, semg, semw, *, cmax, tail, base):
    pltpu.sync_copy(tlm_hbm, tailm.at[pl.ds(0, tail * MLP_DIM)])
    pltpu.sync_copy(tlg_hbm, tailg.at[pl.ds(0, tail * FACTOR)])

    pltpu.sync_copy(sidx_hbm.at[pl.ds(base, BPW)], vtmp)
    pltpu.sync_copy(perm_hbm.at[pl.ds(base, BPW)], vtmp2)

    def smem_fill(k, carry):
        vs = vtmp[pl.ds(k * 16, 16)]
        vp = vtmp2[pl.ds(k * 16, 16)]
        for t in range(16):
            sj[k * 16 + t] = vs[t]
            pj[k * 16 + t] = vp[t]
        return carry

    lax.fori_loop(0, BPW // 16, smem_fill, 0)

    def col_of(k):
        return lax.shift_right_logical(sj[k], 7)

    runs[0] = 0

    def scan_body(j, st):
        r, c = st
        cj = col_of(j)
        is_new = cj != c

        @pl.when(is_new)
        def _():
            runs[r] = j

        return r + jnp.where(is_new, 1, 0), cj

    nrun, _ = lax.fori_loop(1, BPW, scan_body, (jnp.int32(1), col_of(0)))
    runs[nrun] = BPW

    def dmas(c, p):
        lanes = pl.ds(c * 128, 128)
        return (
            pltpu.make_async_copy(
                t2m.at[:, lanes], bufm.at[pl.ds(p * MLP_DIM, MLP_DIM)], semm),
            pltpu.make_async_copy(
                t2g.at[:, lanes], bufg.at[pl.ds(p * FACTOR, FACTOR)], semg),
        )

    def fire(r):
        c = col_of(runs[r])

        @pl.when(c != cmax)
        def _():
            for cp in dmas(c, lax.rem(r, NB)):
                cp.start()

    def drain(c, p):
        @pl.when(c != cmax)
        def _():
            for cp in dmas(c, p):
                cp.wait()

    iota16 = lax.iota(jnp.int32, 16)

    def extract(j, p):
        lvec = jnp.full((16,), lax.rem(sj[j], 128), jnp.int32)
        pvec = jnp.full((16,), p, jnp.int32)
        c = col_of(j)

        @pl.when(c != cmax)
        def _():
            for q in range(MLP_DIM // 16):
                f = iota16 + (16 * q)
                row = plsc.load_gather(bufm, [pvec * MLP_DIM + f, lvec])
                stage[pl.ds(j * 96 + 16 * q, 16)] = row
            for q in range(FACTOR // 16):
                f = iota16 + (16 * q)
                row = plsc.load_gather(bufg, [pvec * FACTOR + f, lvec])
                stage[pl.ds(j * 96 + MLP_DIM + 16 * q, 16)] = row

        @pl.when(c == cmax)
        def _():
            for q in range(MLP_DIM // 16):
                f = iota16 + (16 * q)
                row = plsc.load_gather(tailm, [lvec * MLP_DIM + f])
                stage[pl.ds(j * 96 + 16 * q, 16)] = row
            for q in range(FACTOR // 16):
                f = iota16 + (16 * q)
                row = plsc.load_gather(tailg, [lvec * FACTOR + f])
                stage[pl.ds(j * 96 + MLP_DIM + 16 * q, 16)] = row

    lax.fori_loop(0, jnp.minimum(NB - 1, nrun), lambda r, c: (fire(r), c)[1], 0)

    def run_body(r, carry):
        p = lax.rem(r, NB)
        drain(col_of(runs[r]), p)

        @pl.when(r + NB - 1 < nrun)
        def _():
            fire(r + NB - 1)

        def ext(k, c2):
            extract(k, p)
            return c2

        lax.fori_loop(runs[r], runs[r + 1], ext, 0)
        return carry

    lax.fori_loop(0, nrun, run_body, 0)

    def flush(k, carry):
        handles = []
        for t in range(16):
            j = k * 16 + t
            cp = pltpu.make_async_copy(
                stage.at[pl.ds(j * 96, 96)], out_flat.at[pl.ds(pj[j] * 96, 96)],
                semw)
            cp.start()
            handles.append(cp)
        for cp in handles:
            cp.wait()
        return carry

    lax.fori_loop(0, BPW // 16, flush, 0)


def _sc_body(su, pu, si, pi, t2um, t2ug, t2im, t2ig, tlum, tlug, tlim, tlig,
             o_user, o_item, sj, pj, runs, vtmp, vtmp2, bufm, bufg, stage,
             tailm, tailg, semm, semg, semw):
    wid = lax.axis_index("s") * NC + lax.axis_index("c")
    base = wid * BPW
    _phase(t2um, t2ug, tlum, tlug, su, pu, o_user,
           sj, pj, runs, vtmp, vtmp2, bufm, bufg, stage, tailm,
           tailg, semm, semg, semw, cmax=CMAXU, tail=TAILU, base=base)
    _phase(t2im, t2ig, tlim, tlig, si, pi, o_item,
           sj, pj, runs, vtmp, vtmp2, bufm, bufg, stage, tailm,
           tailg, semm, semg, semw, cmax=CMAXI, tail=TAILI, base=base)


@functools.cache
def _sc_gather():
    return pl.kernel(
        _sc_body,
        out_type=[
            jax.ShapeDtypeStruct((B * 96,), jnp.float32),
            jax.ShapeDtypeStruct((B * 96,), jnp.float32),
        ],
        mesh=plsc.VectorSubcoreMesh(core_axis_name="c", subcore_axis_name="s",
                                    num_cores=NC, num_subcores=NS),
        compiler_params=pltpu.CompilerParams(needs_layout_passes=False),
        scratch_types=[
            pltpu.SMEM((BPW,), jnp.int32),
            pltpu.SMEM((BPW,), jnp.int32),
            pltpu.SMEM((BPW + 2,), jnp.int32),
            pltpu.VMEM((BPW,), jnp.int32),
            pltpu.VMEM((BPW,), jnp.int32),
            pltpu.VMEM((NB * MLP_DIM, 128), jnp.float32),
            pltpu.VMEM((NB * FACTOR, 128), jnp.float32),
            pltpu.VMEM((BPW * 96,), jnp.float32),
            pltpu.VMEM((TAILU * MLP_DIM,), jnp.float32),
            pltpu.VMEM((TAILU * FACTOR,), jnp.float32),
            pltpu.SemaphoreType.DMA,
            pltpu.SemaphoreType.DMA,
            pltpu.SemaphoreType.DMA,
        ],
    )


BLK = 2048


def _dense_body(u_ref, i_ref, w0a_ref, w0b_ref, b0_ref,
                w1_ref, b1_ref, wpa_ref, wpb_ref, bp_ref, out_ref):
    ub = u_ref[...]
    ib = i_ref[...]
    um = ub[:, :MLP_DIM]
    ug = ub[:, MLP_DIM:MLP_DIM + FACTOR]
    im = ib[:, :MLP_DIM]
    ig = ib[:, MLP_DIM:MLP_DIM + FACTOR]
    h = jnp.dot(um, w0a_ref[...], preferred_element_type=jnp.float32)
    h += jnp.dot(im, w0b_ref[...], preferred_element_type=jnp.float32)
    h = jnp.maximum(h + b0_ref[...], 0.0)
    m = jnp.maximum(
        jnp.dot(h, w1_ref[...], preferred_element_type=jnp.float32)
        + b1_ref[...], 0.0)
    g = ug * ig
    out_ref[...] = (jnp.sum(m * wpb_ref[...], axis=1)
                    + jnp.sum(g * wpa_ref[...], axis=1) + bp_ref[...][0, 0])


def _dense(u, i, w0a, w0b, b0, w1, b1, wpa, wpb, bp):
    full = lambda r, c_: pl.BlockSpec((r, c_), lambda k: (0, 0))
    return pl.pallas_call(
        _dense_body,
        grid=(B // BLK,),
        in_specs=[
            pl.BlockSpec((BLK, 96), lambda k: (k, 0)),
            pl.BlockSpec((BLK, 96), lambda k: (k, 0)),
            full(MLP_DIM, MLP_DIM),
            full(MLP_DIM, MLP_DIM),
            full(1, MLP_DIM),
            full(MLP_DIM, FACTOR),
            full(1, FACTOR),
            full(1, FACTOR),
            full(1, FACTOR),
            full(1, 1),
        ],
        out_specs=pl.BlockSpec((BLK,), lambda k: (k,)),
        out_shape=jax.ShapeDtypeStruct((B,), jnp.float32),
    )(u, i, w0a, w0b, b0, w1, b1, wpa, wpb, bp)


def kernel(user, item, emb_user_gmf, emb_item_gmf, emb_user_mlp, emb_item_mlp,
           W0, b0, W1, b1, Wp, bp):
    pos = lax.iota(jnp.int32, B)
    su, pu = lax.sort_key_val(user, pos)
    si, pi = lax.sort_key_val(item, pos)
    ou, oi = _sc_gather()(
        su, pu, si, pi,
        emb_user_mlp.T, emb_user_gmf.T, emb_item_mlp.T, emb_item_gmf.T,
        emb_user_mlp[UN - TAILU:].reshape(-1),
        emb_user_gmf[UN - TAILU:].reshape(-1),
        emb_item_mlp[IT - TAILI:].reshape(-1),
        emb_item_gmf[IT - TAILI:].reshape(-1))
    u2 = ou.reshape(B, 96)
    i2 = oi.reshape(B, 96)
    return _dense(u2, i2, W0[:MLP_DIM], W0[MLP_DIM:], b0.reshape(1, MLP_DIM),
                  W1, b1.reshape(1, FACTOR), Wp[:FACTOR].reshape(1, FACTOR),
                  Wp[FACTOR:].reshape(1, FACTOR), bp.reshape(1, 1))


# speedup vs baseline: 1.4204x; 1.0292x over previous
import functools

import jax
import jax.numpy as jnp
from jax import lax
from jax.experimental import pallas as pl
from jax.experimental.pallas import tpu as pltpu
from jax.experimental.pallas import tpu_sc as plsc

B = 16384
FACTOR = 32
MLP_DIM = 64
UN = 1000000
IT = 100000
NC = 2
NS = 16
NW = NC * NS
BPW = B // NW
CMAXU = (UN - 1) // 128
CMAXI = (IT - 1) // 128
TAILU = UN - CMAXU * 128
TAILI = IT - CMAXI * 128
NB = 5


def _phase(t2m, t2g, tlm_hbm, tlg_hbm, sidx_hbm, perm_hbm, out_flat,
           sj, pj, runs, vtmp, vtmp2, bufm, bufg, stage, tailm, tailg,
           semm, semg, semw, *, cmax, tail, base):
    pltpu.sync_copy(tlm_hbm, tailm.at[pl.ds(0, tail * MLP_DIM)])
    pltpu.sync_copy(tlg_hbm, tailg.at[pl.ds(0, tail * FACTOR)])

    pltpu.sync_copy(sidx_hbm.at[pl.ds(base, BPW)], vtmp)
    pltpu.sync_copy(perm_hbm.at[pl.ds(base, BPW)], vtmp2)

    def smem_fill(k, carry):
        vs = vtmp[pl.ds(k * 16, 16)]
        vp = vtmp2[pl.ds(k * 16, 16)]
        for t in range(16):
            sj[k * 16 + t] = vs[t]
            pj[k * 16 + t] = vp[t]
        return carry

    lax.fori_loop(0, BPW // 16, smem_fill, 0)

    def col_of(k):
        return lax.shift_right_logical(sj[k], 7)

    runs[0] = 0

    def scan_body(j, st):
        r, c = st
        cj = col_of(j)
        is_new = cj != c

        @pl.when(is_new)
        def _():
            runs[r] = j

        return r + jnp.where(is_new, 1, 0), cj

    nrun, _ = lax.fori_loop(1, BPW, scan_body, (jnp.int32(1), col_of(0)))
    runs[nrun] = BPW

    def dmas(c, p):
        lanes = pl.ds(c * 128, 128)
        return (
            pltpu.make_async_copy(
                t2m.at[:, lanes], bufm.at[pl.ds(p * MLP_DIM, MLP_DIM)], semm),
            pltpu.make_async_copy(
                t2g.at[:, lanes], bufg.at[pl.ds(p * FACTOR, FACTOR)], semg),
        )

    def fire(r):
        c = col_of(runs[r])

        @pl.when(c != cmax)
        def _():
            for cp in dmas(c, lax.rem(r, NB)):
                cp.start()

    def drain(c, p):
        @pl.when(c != cmax)
        def _():
            for cp in dmas(c, p):
                cp.wait()

    iota16 = lax.iota(jnp.int32, 16)

    def extract(j, p):
        lvec = jnp.full((16,), lax.rem(sj[j], 128), jnp.int32)
        pvec = jnp.full((16,), p, jnp.int32)
        c = col_of(j)

        @pl.when(c != cmax)
        def _():
            for q in range(MLP_DIM // 16):
                f = iota16 + (16 * q)
                row = plsc.load_gather(bufm, [pvec * MLP_DIM + f, lvec])
                stage[pl.ds(j * 96 + 16 * q, 16)] = row
            for q in range(FACTOR // 16):
                f = iota16 + (16 * q)
                row = plsc.load_gather(bufg, [pvec * FACTOR + f, lvec])
                stage[pl.ds(j * 96 + MLP_DIM + 16 * q, 16)] = row

        @pl.when(c == cmax)
        def _():
            for q in range(MLP_DIM // 16):
                f = iota16 + (16 * q)
                row = plsc.load_gather(tailm, [lvec * MLP_DIM + f])
                stage[pl.ds(j * 96 + 16 * q, 16)] = row
            for q in range(FACTOR // 16):
                f = iota16 + (16 * q)
                row = plsc.load_gather(tailg, [lvec * FACTOR + f])
                stage[pl.ds(j * 96 + MLP_DIM + 16 * q, 16)] = row

    lax.fori_loop(0, jnp.minimum(NB - 1, nrun), lambda r, c: (fire(r), c)[1], 0)

    def run_body(r, carry):
        p = lax.rem(r, NB)
        drain(col_of(runs[r]), p)

        @pl.when(r + NB - 1 < nrun)
        def _():
            fire(r + NB - 1)

        def ext(k, c2):
            extract(k, p)
            return c2

        lax.fori_loop(runs[r], runs[r + 1], ext, 0)
        return carry

    lax.fori_loop(0, nrun, run_body, 0)

    def flush(k, carry):
        handles = []
        for t in range(16):
            j = k * 16 + t
            cp = pltpu.make_async_copy(
                stage.at[pl.ds(j * 96, 96)], out_flat.at[pl.ds(pj[j] * 96, 96)],
                semw)
            cp.start()
            handles.append(cp)
        for cp in handles:
            cp.wait()
        return carry

    lax.fori_loop(0, BPW // 16, flush, 0)


def _sc_body(su, pu, si, pi, t2um, t2ug, t2im, t2ig, tlum, tlug, tlim, tlig,
             o_user, o_item, sj, pj, runs, vtmp, vtmp2, bufm, bufg, stage,
             tailm, tailg, semm, semg, semw):
    wid = lax.axis_index("s") * NC + lax.axis_index("c")
    base = wid * BPW
    _phase(t2um, t2ug, tlum, tlug, su, pu, o_user,
           sj, pj, runs, vtmp, vtmp2, bufm, bufg, stage, tailm,
           tailg, semm, semg, semw, cmax=CMAXU, tail=TAILU, base=base)
    _phase(t2im, t2ig, tlim, tlig, si, pi, o_item,
           sj, pj, runs, vtmp, vtmp2, bufm, bufg, stage, tailm,
           tailg, semm, semg, semw, cmax=CMAXI, tail=TAILI, base=base)


@functools.cache
def _sc_gather():
    return pl.kernel(
        _sc_body,
        out_type=[
            jax.ShapeDtypeStruct((B * 96,), jnp.float32),
            jax.ShapeDtypeStruct((B * 96,), jnp.float32),
        ],
        mesh=plsc.VectorSubcoreMesh(core_axis_name="c", subcore_axis_name="s",
                                    num_cores=NC, num_subcores=NS),
        compiler_params=pltpu.CompilerParams(needs_layout_passes=False),
        scratch_types=[
            pltpu.SMEM((BPW,), jnp.int32),
            pltpu.SMEM((BPW,), jnp.int32),
            pltpu.SMEM((BPW + 2,), jnp.int32),
            pltpu.VMEM((BPW,), jnp.int32),
            pltpu.VMEM((BPW,), jnp.int32),
            pltpu.VMEM((NB * MLP_DIM, 128), jnp.float32),
            pltpu.VMEM((NB * FACTOR, 128), jnp.float32),
            pltpu.VMEM((BPW * 96,), jnp.float32),
            pltpu.VMEM((TAILU * MLP_DIM,), jnp.float32),
            pltpu.VMEM((TAILU * FACTOR,), jnp.float32),
            pltpu.SemaphoreType.DMA,
            pltpu.SemaphoreType.DMA,
            pltpu.SemaphoreType.DMA,
        ],
    )


BLK = 2048


def _dense_body(u_ref, i_ref, w0a_ref, w0b_ref, b0_ref,
                w1_ref, b1_ref, wpa_ref, wpb_ref, bp_ref, out_ref):
    ub = u_ref[...]
    ib = i_ref[...]
    um = ub[:, :MLP_DIM]
    ug = ub[:, MLP_DIM:MLP_DIM + FACTOR]
    im = ib[:, :MLP_DIM]
    ig = ib[:, MLP_DIM:MLP_DIM + FACTOR]
    h = jnp.dot(um, w0a_ref[...], preferred_element_type=jnp.float32)
    h += jnp.dot(im, w0b_ref[...], preferred_element_type=jnp.float32)
    h = jnp.maximum(h + b0_ref[...], 0.0)
    m = jnp.maximum(
        jnp.dot(h, w1_ref[...], preferred_element_type=jnp.float32)
        + b1_ref[...], 0.0)
    g = ug * ig
    out_ref[...] = (jnp.sum(m * wpb_ref[...], axis=1)
                    + jnp.sum(g * wpa_ref[...], axis=1) + bp_ref[...][0, 0])


def _dense(u, i, w0a, w0b, b0, w1, b1, wpa, wpb, bp):
    full = lambda r, c_: pl.BlockSpec((r, c_), lambda k: (0, 0))
    return pl.pallas_call(
        _dense_body,
        grid=(B // BLK,),
        in_specs=[
            pl.BlockSpec((BLK, 96), lambda k: (k, 0)),
            pl.BlockSpec((BLK, 96), lambda k: (k, 0)),
            full(MLP_DIM, MLP_DIM),
            full(MLP_DIM, MLP_DIM),
            full(1, MLP_DIM),
            full(MLP_DIM, FACTOR),
            full(1, FACTOR),
            full(1, FACTOR),
            full(1, FACTOR),
            full(1, 1),
        ],
        out_specs=pl.BlockSpec((BLK,), lambda k: (k,)),
        out_shape=jax.ShapeDtypeStruct((B,), jnp.float32),
    )(u, i, w0a, w0b, b0, w1, b1, wpa, wpb, bp)


def kernel(user, item, emb_user_gmf, emb_item_gmf, emb_user_mlp, emb_item_mlp,
           W0, b0, W1, b1, Wp, bp):
    pos = lax.iota(jnp.int32, B)
    su, pu = lax.sort_key_val(user, pos)
    si, pi = lax.sort_key_val(item, pos)
    ou, oi = _sc_gather()(
        su, pu, si, pi,
        emb_user_mlp.T, emb_user_gmf.T, emb_item_mlp.T, emb_item_gmf.T,
        emb_user_mlp[UN - TAILU:].reshape(-1),
        emb_user_gmf[UN - TAILU:].reshape(-1),
        emb_item_mlp[IT - TAILI:].reshape(-1),
        emb_item_gmf[IT - TAILI:].reshape(-1))
    u2 = ou.reshape(B, 96)
    i2 = oi.reshape(B, 96)
    return _dense(u2, i2, W0[:MLP_DIM], W0[MLP_DIM:], b0.reshape(1, MLP_DIM),
                  W1, b1.reshape(1, FACTOR), Wp[:FACTOR].reshape(1, FACTOR),
                  Wp[FACTOR:].reshape(1, FACTOR), bp.reshape(1, 1))


# speedup vs baseline: 1.5367x; 1.0819x over previous
import functools

import jax
import jax.numpy as jnp
from jax import lax
from jax.experimental import pallas as pl
from jax.experimental.pallas import tpu as pltpu
from jax.experimental.pallas import tpu_sc as plsc

B = 16384
FACTOR = 32
MLP_DIM = 64
UN = 1000000
IT = 100000
NC = 2
NS = 16
NW = NC * NS
BPW = B // NW
CMAXU = (UN - 1) // 128
CMAXI = (IT - 1) // 128
TAILU = UN - CMAXU * 128
TAILI = IT - CMAXI * 128
NB = 5


def _phase(t2m, t2g, tlm_hbm, tlg_hbm, sidx_hbm, perm_hbm, out_flat,
           sj, pj, runs, vtmp, vtmp2, bufm, bufg, stage, tailm, tailg,
           semm, semg, semw, *, cmax, tail, base):
    pltpu.sync_copy(tlm_hbm, tailm.at[pl.ds(0, tail * MLP_DIM)])
    pltpu.sync_copy(tlg_hbm, tailg.at[pl.ds(0, tail * FACTOR)])

    pltpu.sync_copy(sidx_hbm.at[pl.ds(base, BPW)], vtmp)
    pltpu.sync_copy(perm_hbm.at[pl.ds(base, BPW)], vtmp2)

    def smem_fill(k, carry):
        vs = vtmp[pl.ds(k * 16, 16)]
        vp = vtmp2[pl.ds(k * 16, 16)]
        for t in range(16):
            sj[k * 16 + t] = vs[t]
            pj[k * 16 + t] = vp[t]
        return carry

    lax.fori_loop(0, BPW // 16, smem_fill, 0)

    def col_of(k):
        return lax.shift_right_logical(sj[k], 7)

    runs[0] = 0

    def scan_body(j, st):
        r, c = st
        cj = col_of(j)
        is_new = cj != c

        @pl.when(is_new)
        def _():
            runs[r] = j

        return r + jnp.where(is_new, 1, 0), cj

    nrun, _ = lax.fori_loop(1, BPW, scan_body, (jnp.int32(1), col_of(0)))
    runs[nrun] = BPW

    def dmas(c, p):
        lanes = pl.ds(c * 128, 128)
        return (
            pltpu.make_async_copy(
                t2m.at[:, lanes], bufm.at[pl.ds(p * MLP_DIM, MLP_DIM)], semm),
            pltpu.make_async_copy(
                t2g.at[:, lanes], bufg.at[pl.ds(p * FACTOR, FACTOR)], semg),
        )

    def fire(r):
        c = col_of(runs[r])

        @pl.when(c != cmax)
        def _():
            for cp in dmas(c, lax.rem(r, NB)):
                cp.start()

    def drain(c, p):
        @pl.when(c != cmax)
        def _():
            for cp in dmas(c, p):
                cp.wait()

    iota16 = lax.iota(jnp.int32, 16)

    def extract(j, p):
        lvec = jnp.full((16,), lax.rem(sj[j], 128), jnp.int32)
        pvec = jnp.full((16,), p, jnp.int32)
        c = col_of(j)

        @pl.when(c != cmax)
        def _():
            for q in range(MLP_DIM // 16):
                f = iota16 + (16 * q)
                row = plsc.load_gather(bufm, [pvec * MLP_DIM + f, lvec])
                stage[pl.ds(j * 96 + 16 * q, 16)] = row
            for q in range(FACTOR // 16):
                f = iota16 + (16 * q)
                row = plsc.load_gather(bufg, [pvec * FACTOR + f, lvec])
                stage[pl.ds(j * 96 + MLP_DIM + 16 * q, 16)] = row

        @pl.when(c == cmax)
        def _():
            for q in range(MLP_DIM // 16):
                f = iota16 + (16 * q)
                row = plsc.load_gather(tailm, [lvec * MLP_DIM + f])
                stage[pl.ds(j * 96 + 16 * q, 16)] = row
            for q in range(FACTOR // 16):
                f = iota16 + (16 * q)
                row = plsc.load_gather(tailg, [lvec * FACTOR + f])
                stage[pl.ds(j * 96 + MLP_DIM + 16 * q, 16)] = row

    lax.fori_loop(0, jnp.minimum(NB - 1, nrun), lambda r, c: (fire(r), c)[1], 0)

    def run_body(r, carry):
        p = lax.rem(r, NB)
        drain(col_of(runs[r]), p)

        @pl.when(r + NB - 1 < nrun)
        def _():
            fire(r + NB - 1)

        def ext(k, c2):
            extract(k, p)
            return c2

        lax.fori_loop(runs[r], runs[r + 1], ext, 0)
        return carry

    lax.fori_loop(0, nrun, run_body, 0)

    def flush(k, carry):
        handles = []
        for t in range(16):
            j = k * 16 + t
            cp = pltpu.make_async_copy(
                stage.at[pl.ds(j * 96, 96)],
                out_flat.at[pl.ds(pj[j] * 128, 96)], semw)
            cp.start()
            handles.append(cp)
        for cp in handles:
            cp.wait()
        return carry

    lax.fori_loop(0, BPW // 16, flush, 0)


def _sc_body(su, pu, si, pi, t2um, t2ug, t2im, t2ig, tlum, tlug, tlim, tlig,
             o_user, o_item, sj, pj, runs, vtmp, vtmp2, bufm, bufg, stage,
             tailm, tailg, semm, semg, semw):
    wid = lax.axis_index("s") * NC + lax.axis_index("c")
    base = wid * BPW
    _phase(t2um, t2ug, tlum, tlug, su, pu, o_user,
           sj, pj, runs, vtmp, vtmp2, bufm, bufg, stage, tailm,
           tailg, semm, semg, semw, cmax=CMAXU, tail=TAILU, base=base)
    _phase(t2im, t2ig, tlim, tlig, si, pi, o_item,
           sj, pj, runs, vtmp, vtmp2, bufm, bufg, stage, tailm,
           tailg, semm, semg, semw, cmax=CMAXI, tail=TAILI, base=base)


@functools.cache
def _sc_gather():
    return pl.kernel(
        _sc_body,
        out_type=[
            jax.ShapeDtypeStruct((B * 128,), jnp.float32),
            jax.ShapeDtypeStruct((B * 128,), jnp.float32),
        ],
        mesh=plsc.VectorSubcoreMesh(core_axis_name="c", subcore_axis_name="s",
                                    num_cores=NC, num_subcores=NS),
        compiler_params=pltpu.CompilerParams(needs_layout_passes=False),
        scratch_types=[
            pltpu.SMEM((BPW,), jnp.int32),
            pltpu.SMEM((BPW,), jnp.int32),
            pltpu.SMEM((BPW + 2,), jnp.int32),
            pltpu.VMEM((BPW,), jnp.int32),
            pltpu.VMEM((BPW,), jnp.int32),
            pltpu.VMEM((NB * MLP_DIM, 128), jnp.float32),
            pltpu.VMEM((NB * FACTOR, 128), jnp.float32),
            pltpu.VMEM((BPW * 96,), jnp.float32),
            pltpu.VMEM((TAILU * MLP_DIM,), jnp.float32),
            pltpu.VMEM((TAILU * FACTOR,), jnp.float32),
            pltpu.SemaphoreType.DMA,
            pltpu.SemaphoreType.DMA,
            pltpu.SemaphoreType.DMA,
        ],
    )


BLK = 2048


def _dense_body(u_ref, i_ref, w0a_ref, w0b_ref, b0_ref,
                w1_ref, b1_ref, wpa_ref, wpb_ref, bp_ref, out_ref):
    ub = u_ref[...]
    ib = i_ref[...]
    um = ub[:, :MLP_DIM]
    ug = ub[:, MLP_DIM:MLP_DIM + FACTOR]
    im = ib[:, :MLP_DIM]
    ig = ib[:, MLP_DIM:MLP_DIM + FACTOR]
    h = jnp.dot(um, w0a_ref[...], preferred_element_type=jnp.float32)
    h += jnp.dot(im, w0b_ref[...], preferred_element_type=jnp.float32)
    h = jnp.maximum(h + b0_ref[...], 0.0)
    m = jnp.maximum(
        jnp.dot(h, w1_ref[...], preferred_element_type=jnp.float32)
        + b1_ref[...], 0.0)
    g = ug * ig
    out_ref[...] = (jnp.sum(m * wpb_ref[...], axis=1)
                    + jnp.sum(g * wpa_ref[...], axis=1) + bp_ref[...][0, 0])


def _dense(u, i, w0a, w0b, b0, w1, b1, wpa, wpb, bp):
    full = lambda r, c_: pl.BlockSpec((r, c_), lambda k: (0, 0))
    return pl.pallas_call(
        _dense_body,
        grid=(B // BLK,),
        in_specs=[
            pl.BlockSpec((BLK, 128), lambda k: (k, 0)),
            pl.BlockSpec((BLK, 128), lambda k: (k, 0)),
            full(MLP_DIM, MLP_DIM),
            full(MLP_DIM, MLP_DIM),
            full(1, MLP_DIM),
            full(MLP_DIM, FACTOR),
            full(1, FACTOR),
            full(1, FACTOR),
            full(1, FACTOR),
            full(1, 1),
        ],
        out_specs=pl.BlockSpec((BLK,), lambda k: (k,)),
        out_shape=jax.ShapeDtypeStruct((B,), jnp.float32),
    )(u, i, w0a, w0b, b0, w1, b1, wpa, wpb, bp)


def kernel(user, item, emb_user_gmf, emb_item_gmf, emb_user_mlp, emb_item_mlp,
           W0, b0, W1, b1, Wp, bp):
    pos = lax.iota(jnp.int32, B)
    su, pu = lax.sort_key_val(user, pos)
    si, pi = lax.sort_key_val(item, pos)
    ou, oi = _sc_gather()(
        su, pu, si, pi,
        emb_user_mlp.T, emb_user_gmf.T, emb_item_mlp.T, emb_item_gmf.T,
        emb_user_mlp[UN - TAILU:].reshape(-1),
        emb_user_gmf[UN - TAILU:].reshape(-1),
        emb_item_mlp[IT - TAILI:].reshape(-1),
        emb_item_gmf[IT - TAILI:].reshape(-1))
    u2 = ou.reshape(B, 128)
    i2 = oi.reshape(B, 128)
    return _dense(u2, i2, W0[:MLP_DIM], W0[MLP_DIM:], b0.reshape(1, MLP_DIM),
                  W1, b1.reshape(1, FACTOR), Wp[:FACTOR].reshape(1, FACTOR),
                  Wp[FACTOR:].reshape(1, FACTOR), bp.reshape(1, 1))


# speedup vs baseline: 1.5664x; 1.0193x over previous
import functools

import jax
import jax.numpy as jnp
from jax import lax
from jax.experimental import pallas as pl
from jax.experimental.pallas import tpu as pltpu
from jax.experimental.pallas import tpu_sc as plsc

B = 16384
FACTOR = 32
MLP_DIM = 64
UN = 1000000
IT = 100000
NC = 2
NS = 16
NW = NC * NS
BPW = B // NW
CMAXU = (UN - 1) // 128
CMAXI = (IT - 1) // 128
TAILU = UN - CMAXU * 128
TAILI = IT - CMAXI * 128
NB = 5


def _phase(t2m, t2g, tlm_hbm, tlg_hbm, sidx_hbm, perm_hbm, out_flat,
           sj, pj, runs, vtmp, vtmp2, bufm, bufg, stage, tailm, tailg,
           semm, semg, semw, *, cmax, tail, base):
    pltpu.sync_copy(tlm_hbm, tailm.at[pl.ds(0, tail * MLP_DIM)])
    pltpu.sync_copy(tlg_hbm, tailg.at[pl.ds(0, tail * FACTOR)])

    pltpu.sync_copy(sidx_hbm.at[pl.ds(base, BPW)], vtmp)
    pltpu.sync_copy(perm_hbm.at[pl.ds(base, BPW)], vtmp2)

    def smem_fill(k, carry):
        vs = vtmp[pl.ds(k * 16, 16)]
        vp = vtmp2[pl.ds(k * 16, 16)]
        for t in range(16):
            sj[k * 16 + t] = vs[t]
            pj[k * 16 + t] = vp[t]
        return carry

    lax.fori_loop(0, BPW // 16, smem_fill, 0)

    def col_of(k):
        return lax.shift_right_logical(sj[k], 7)

    runs[0] = 0

    def scan_body(j, st):
        r, c = st
        cj = col_of(j)
        is_new = cj != c

        @pl.when(is_new)
        def _():
            runs[r] = j

        return r + jnp.where(is_new, 1, 0), cj

    nrun, _ = lax.fori_loop(1, BPW, scan_body, (jnp.int32(1), col_of(0)))
    runs[nrun] = BPW

    def dmas(c, p):
        lanes = pl.ds(c * 128, 128)
        return (
            pltpu.make_async_copy(
                t2m.at[:, lanes], bufm.at[pl.ds(p * MLP_DIM, MLP_DIM)], semm),
            pltpu.make_async_copy(
                t2g.at[:, lanes], bufg.at[pl.ds(p * FACTOR, FACTOR)], semg),
        )

    def fire(r):
        c = col_of(runs[r])

        @pl.when(c != cmax)
        def _():
            for cp in dmas(c, lax.rem(r, NB)):
                cp.start()

    def drain(c, p):
        @pl.when(c != cmax)
        def _():
            for cp in dmas(c, p):
                cp.wait()

    iota16 = lax.iota(jnp.int32, 16)

    def extract(j, p):
        lvec = jnp.full((16,), lax.rem(sj[j], 128), jnp.int32)
        pvec = jnp.full((16,), p, jnp.int32)
        c = col_of(j)

        @pl.when(c != cmax)
        def _():
            for q in range(MLP_DIM // 16):
                f = iota16 + (16 * q)
                row = plsc.load_gather(bufm, [pvec * MLP_DIM + f, lvec])
                stage[pl.ds(j * 96 + 16 * q, 16)] = row
            for q in range(FACTOR // 16):
                f = iota16 + (16 * q)
                row = plsc.load_gather(bufg, [pvec * FACTOR + f, lvec])
                stage[pl.ds(j * 96 + MLP_DIM + 16 * q, 16)] = row

        @pl.when(c == cmax)
        def _():
            for q in range(MLP_DIM // 16):
                f = iota16 + (16 * q)
                row = plsc.load_gather(tailm, [lvec * MLP_DIM + f])
                stage[pl.ds(j * 96 + 16 * q, 16)] = row
            for q in range(FACTOR // 16):
                f = iota16 + (16 * q)
                row = plsc.load_gather(tailg, [lvec * FACTOR + f])
                stage[pl.ds(j * 96 + MLP_DIM + 16 * q, 16)] = row

    lax.fori_loop(0, jnp.minimum(NB - 1, nrun), lambda r, c: (fire(r), c)[1], 0)

    def run_body(r, carry):
        p = lax.rem(r, NB)
        drain(col_of(runs[r]), p)

        @pl.when(r + NB - 1 < nrun)
        def _():
            fire(r + NB - 1)

        def ext(k, c2):
            extract(k, p)
            return c2

        lax.fori_loop(runs[r], runs[r + 1], ext, 0)
        return carry

    lax.fori_loop(0, nrun, run_body, 0)

    def flush(k, carry):
        handles = []
        for t in range(16):
            j = k * 16 + t
            cp = pltpu.make_async_copy(
                stage.at[pl.ds(j * 96, 96)],
                out_flat.at[pl.ds(pj[j] * 128, 96)], semw)
            cp.start()
            handles.append(cp)
        for cp in handles:
            cp.wait()
        return carry

    lax.fori_loop(0, BPW // 16, flush, 0)


def _make_sc_body(cmax, tail):
    def _sc_body(sidx, perm, t2m, t2g, tlm, tlg, out_flat,
                 sj, pj, runs, vtmp, vtmp2, bufm, bufg, stage,
                 tailm, tailg, semm, semg, semw):
        wid = lax.axis_index("s") * NC + lax.axis_index("c")
        base = wid * BPW
        _phase(t2m, t2g, tlm, tlg, sidx, perm, out_flat,
               sj, pj, runs, vtmp, vtmp2, bufm, bufg, stage, tailm,
               tailg, semm, semg, semw, cmax=cmax, tail=tail, base=base)

    return _sc_body


@functools.cache
def _sc_gather(cmax, tail):
    return pl.kernel(
        _make_sc_body(cmax, tail),
        out_type=jax.ShapeDtypeStruct((B * 128,), jnp.float32),
        mesh=plsc.VectorSubcoreMesh(core_axis_name="c", subcore_axis_name="s",
                                    num_cores=NC, num_subcores=NS),
        compiler_params=pltpu.CompilerParams(needs_layout_passes=False),
        scratch_types=[
            pltpu.SMEM((BPW,), jnp.int32),
            pltpu.SMEM((BPW,), jnp.int32),
            pltpu.SMEM((BPW + 2,), jnp.int32),
            pltpu.VMEM((BPW,), jnp.int32),
            pltpu.VMEM((BPW,), jnp.int32),
            pltpu.VMEM((NB * MLP_DIM, 128), jnp.float32),
            pltpu.VMEM((NB * FACTOR, 128), jnp.float32),
            pltpu.VMEM((BPW * 96,), jnp.float32),
            pltpu.VMEM((TAILU * MLP_DIM,), jnp.float32),
            pltpu.VMEM((TAILU * FACTOR,), jnp.float32),
            pltpu.SemaphoreType.DMA,
            pltpu.SemaphoreType.DMA,
            pltpu.SemaphoreType.DMA,
        ],
    )


BLK = 2048


def _dense_body(u_ref, i_ref, w0a_ref, w0b_ref, b0_ref,
                w1_ref, b1_ref, wpa_ref, wpb_ref, bp_ref, out_ref):
    ub = u_ref[...]
    ib = i_ref[...]
    um = ub[:, :MLP_DIM]
    ug = ub[:, MLP_DIM:MLP_DIM + FACTOR]
    im = ib[:, :MLP_DIM]
    ig = ib[:, MLP_DIM:MLP_DIM + FACTOR]
    h = jnp.dot(um, w0a_ref[...], preferred_element_type=jnp.float32)
    h += jnp.dot(im, w0b_ref[...], preferred_element_type=jnp.float32)
    h = jnp.maximum(h + b0_ref[...], 0.0)
    m = jnp.maximum(
        jnp.dot(h, w1_ref[...], preferred_element_type=jnp.float32)
        + b1_ref[...], 0.0)
    g = ug * ig
    out_ref[...] = (jnp.sum(m * wpb_ref[...], axis=1)
                    + jnp.sum(g * wpa_ref[...], axis=1) + bp_ref[...][0, 0])


def _dense(u, i, w0a, w0b, b0, w1, b1, wpa, wpb, bp):
    full = lambda r, c_: pl.BlockSpec((r, c_), lambda k: (0, 0))
    return pl.pallas_call(
        _dense_body,
        grid=(B // BLK,),
        in_specs=[
            pl.BlockSpec((BLK, 128), lambda k: (k, 0)),
            pl.BlockSpec((BLK, 128), lambda k: (k, 0)),
            full(MLP_DIM, MLP_DIM),
            full(MLP_DIM, MLP_DIM),
            full(1, MLP_DIM),
            full(MLP_DIM, FACTOR),
            full(1, FACTOR),
            full(1, FACTOR),
            full(1, FACTOR),
            full(1, 1),
        ],
        out_specs=pl.BlockSpec((BLK,), lambda k: (k,)),
        out_shape=jax.ShapeDtypeStruct((B,), jnp.float32),
    )(u, i, w0a, w0b, b0, w1, b1, wpa, wpb, bp)


def kernel(user, item, emb_user_gmf, emb_item_gmf, emb_user_mlp, emb_item_mlp,
           W0, b0, W1, b1, Wp, bp):
    pos = lax.iota(jnp.int32, B)
    su, pu = lax.sort_key_val(user, pos)
    si, pi = lax.sort_key_val(item, pos)
    ou = _sc_gather(CMAXU, TAILU)(
        su, pu, emb_user_mlp.T, emb_user_gmf.T,
        emb_user_mlp[UN - TAILU:].reshape(-1),
        emb_user_gmf[UN - TAILU:].reshape(-1))
    oi = _sc_gather(CMAXI, TAILI)(
        si, pi, emb_item_mlp.T, emb_item_gmf.T,
        emb_item_mlp[IT - TAILI:].reshape(-1),
        emb_item_gmf[IT - TAILI:].reshape(-1))
    u2 = ou.reshape(B, 128)
    i2 = oi.reshape(B, 128)
    return _dense(u2, i2, W0[:MLP_DIM], W0[MLP_DIM:], b0.reshape(1, MLP_DIM),
                  W1, b1.reshape(1, FACTOR), Wp[:FACTOR].reshape(1, FACTOR),
                  Wp[FACTOR:].reshape(1, FACTOR), bp.reshape(1, 1))


# speedup vs baseline: 1.5822x; 1.0101x over previous
import functools

import jax
import jax.numpy as jnp
from jax import lax
from jax.experimental import pallas as pl
from jax.experimental.pallas import tpu as pltpu
from jax.experimental.pallas import tpu_sc as plsc

B = 16384
FACTOR = 32
MLP_DIM = 64
UN = 1000000
IT = 100000
NC = 2
NS = 16
NW = NC * NS
BPW = B // NW
CMAXU = (UN - 1) // 128
CMAXI = (IT - 1) // 128
TAILU = UN - CMAXU * 128
TAILI = IT - CMAXI * 128
NB = 6


def _phase(t2m, t2g, tlm_hbm, tlg_hbm, sidx_hbm, perm_hbm, out_flat,
           sj, pj, runs, vtmp, vtmp2, bufm, bufg, stage, tailm, tailg,
           semm, semg, semw, *, cmax, tail, base):
    pltpu.sync_copy(tlm_hbm, tailm.at[pl.ds(0, tail * MLP_DIM)])
    pltpu.sync_copy(tlg_hbm, tailg.at[pl.ds(0, tail * FACTOR)])

    pltpu.sync_copy(sidx_hbm.at[pl.ds(base, BPW)], vtmp)
    pltpu.sync_copy(perm_hbm.at[pl.ds(base, BPW)], vtmp2)

    def smem_fill(k, carry):
        vs = vtmp[pl.ds(k * 16, 16)]
        vp = vtmp2[pl.ds(k * 16, 16)]
        for t in range(16):
            sj[k * 16 + t] = vs[t]
            pj[k * 16 + t] = vp[t]
        return carry

    lax.fori_loop(0, BPW // 16, smem_fill, 0)

    def col_of(k):
        return lax.shift_right_logical(sj[k], 7)

    runs[0] = 0

    def scan_body(j, st):
        r, c = st
        cj = col_of(j)
        is_new = cj != c

        @pl.when(is_new)
        def _():
            runs[r] = j

        return r + jnp.where(is_new, 1, 0), cj

    nrun, _ = lax.fori_loop(1, BPW, scan_body, (jnp.int32(1), col_of(0)))
    runs[nrun] = BPW

    def dmas(c, p):
        lanes = pl.ds(c * 128, 128)
        return (
            pltpu.make_async_copy(
                t2m.at[:, lanes], bufm.at[pl.ds(p * MLP_DIM, MLP_DIM)], semm),
            pltpu.make_async_copy(
                t2g.at[:, lanes], bufg.at[pl.ds(p * FACTOR, FACTOR)], semg),
        )

    def fire(r):
        c = col_of(runs[r])

        @pl.when(c != cmax)
        def _():
            for cp in dmas(c, lax.rem(r, NB)):
                cp.start()

    def drain(c, p):
        @pl.when(c != cmax)
        def _():
            for cp in dmas(c, p):
                cp.wait()

    iota16 = lax.iota(jnp.int32, 16)

    def extract(j, p):
        lvec = jnp.full((16,), lax.rem(sj[j], 128), jnp.int32)
        pvec = jnp.full((16,), p, jnp.int32)
        c = col_of(j)

        @pl.when(c != cmax)
        def _():
            for q in range(MLP_DIM // 16):
                f = iota16 + (16 * q)
                row = plsc.load_gather(bufm, [pvec * MLP_DIM + f, lvec])
                stage[pl.ds(j * 96 + 16 * q, 16)] = row
            for q in range(FACTOR // 16):
                f = iota16 + (16 * q)
                row = plsc.load_gather(bufg, [pvec * FACTOR + f, lvec])
                stage[pl.ds(j * 96 + MLP_DIM + 16 * q, 16)] = row

        @pl.when(c == cmax)
        def _():
            for q in range(MLP_DIM // 16):
                f = iota16 + (16 * q)
                row = plsc.load_gather(tailm, [lvec * MLP_DIM + f])
                stage[pl.ds(j * 96 + 16 * q, 16)] = row
            for q in range(FACTOR // 16):
                f = iota16 + (16 * q)
                row = plsc.load_gather(tailg, [lvec * FACTOR + f])
                stage[pl.ds(j * 96 + MLP_DIM + 16 * q, 16)] = row

    lax.fori_loop(0, jnp.minimum(NB - 1, nrun), lambda r, c: (fire(r), c)[1], 0)

    def run_body(r, carry):
        p = lax.rem(r, NB)
        drain(col_of(runs[r]), p)

        @pl.when(r + NB - 1 < nrun)
        def _():
            fire(r + NB - 1)

        def ext(k, c2):
            extract(k, p)
            return c2

        lax.fori_loop(runs[r], runs[r + 1], ext, 0)
        return carry

    lax.fori_loop(0, nrun, run_body, 0)

    def flush(k, carry):
        handles = []
        for t in range(16):
            j = k * 16 + t
            cp = pltpu.make_async_copy(
                stage.at[pl.ds(j * 96, 96)],
                out_flat.at[pl.ds(pj[j] * 128, 96)], semw)
            cp.start()
            handles.append(cp)
        for cp in handles:
            cp.wait()
        return carry

    lax.fori_loop(0, BPW // 16, flush, 0)


def _make_sc_body(cmax, tail):
    def _sc_body(sidx, perm, t2m, t2g, tlm, tlg, out_flat,
                 sj, pj, runs, vtmp, vtmp2, bufm, bufg, stage,
                 tailm, tailg, semm, semg, semw):
        wid = lax.axis_index("s") * NC + lax.axis_index("c")
        base = wid * BPW
        _phase(t2m, t2g, tlm, tlg, sidx, perm, out_flat,
               sj, pj, runs, vtmp, vtmp2, bufm, bufg, stage, tailm,
               tailg, semm, semg, semw, cmax=cmax, tail=tail, base=base)

    return _sc_body


@functools.cache
def _sc_gather(cmax, tail):
    return pl.kernel(
        _make_sc_body(cmax, tail),
        out_type=jax.ShapeDtypeStruct((B * 128,), jnp.float32),
        mesh=plsc.VectorSubcoreMesh(core_axis_name="c", subcore_axis_name="s",
                                    num_cores=NC, num_subcores=NS),
        compiler_params=pltpu.CompilerParams(needs_layout_passes=False),
        scratch_types=[
            pltpu.SMEM((BPW,), jnp.int32),
            pltpu.SMEM((BPW,), jnp.int32),
            pltpu.SMEM((BPW + 2,), jnp.int32),
            pltpu.VMEM((BPW,), jnp.int32),
            pltpu.VMEM((BPW,), jnp.int32),
            pltpu.VMEM((NB * MLP_DIM, 128), jnp.float32),
            pltpu.VMEM((NB * FACTOR, 128), jnp.float32),
            pltpu.VMEM((BPW * 96,), jnp.float32),
            pltpu.VMEM((TAILU * MLP_DIM,), jnp.float32),
            pltpu.VMEM((TAILU * FACTOR,), jnp.float32),
            pltpu.SemaphoreType.DMA,
            pltpu.SemaphoreType.DMA,
            pltpu.SemaphoreType.DMA,
        ],
    )


BLK = 2048


def _dense_body(u_ref, i_ref, w0a_ref, w0b_ref, b0_ref,
                w1_ref, b1_ref, wpa_ref, wpb_ref, bp_ref, out_ref):
    ub = u_ref[...]
    ib = i_ref[...]
    um = ub[:, :MLP_DIM]
    ug = ub[:, MLP_DIM:MLP_DIM + FACTOR]
    im = ib[:, :MLP_DIM]
    ig = ib[:, MLP_DIM:MLP_DIM + FACTOR]
    h = jnp.dot(um, w0a_ref[...], preferred_element_type=jnp.float32)
    h += jnp.dot(im, w0b_ref[...], preferred_element_type=jnp.float32)
    h = jnp.maximum(h + b0_ref[...], 0.0)
    m = jnp.maximum(
        jnp.dot(h, w1_ref[...], preferred_element_type=jnp.float32)
        + b1_ref[...], 0.0)
    g = ug * ig
    out_ref[...] = (jnp.sum(m * wpb_ref[...], axis=1)
                    + jnp.sum(g * wpa_ref[...], axis=1) + bp_ref[...][0, 0])


def _dense(u, i, w0a, w0b, b0, w1, b1, wpa, wpb, bp):
    full = lambda r, c_: pl.BlockSpec((r, c_), lambda k: (0, 0))
    return pl.pallas_call(
        _dense_body,
        grid=(B // BLK,),
        in_specs=[
            pl.BlockSpec((BLK, 128), lambda k: (k, 0)),
            pl.BlockSpec((BLK, 128), lambda k: (k, 0)),
            full(MLP_DIM, MLP_DIM),
            full(MLP_DIM, MLP_DIM),
            full(1, MLP_DIM),
            full(MLP_DIM, FACTOR),
            full(1, FACTOR),
            full(1, FACTOR),
            full(1, FACTOR),
            full(1, 1),
        ],
        out_specs=pl.BlockSpec((BLK,), lambda k: (k,)),
        out_shape=jax.ShapeDtypeStruct((B,), jnp.float32),
    )(u, i, w0a, w0b, b0, w1, b1, wpa, wpb, bp)


def kernel(user, item, emb_user_gmf, emb_item_gmf, emb_user_mlp, emb_item_mlp,
           W0, b0, W1, b1, Wp, bp):
    pos = lax.iota(jnp.int32, B)
    su, pu = lax.sort_key_val(user, pos)
    si, pi = lax.sort_key_val(item, pos)
    ou = _sc_gather(CMAXU, TAILU)(
        su, pu, emb_user_mlp.T, emb_user_gmf.T,
        emb_user_mlp[UN - TAILU:].reshape(-1),
        emb_user_gmf[UN - TAILU:].reshape(-1))
    oi = _sc_gather(CMAXI, TAILI)(
        si, pi, emb_item_mlp.T, emb_item_gmf.T,
        emb_item_mlp[IT - TAILI:].reshape(-1),
        emb_item_gmf[IT - TAILI:].reshape(-1))
    u2 = ou.reshape(B, 128)
    i2 = oi.reshape(B, 128)
    return _dense(u2, i2, W0[:MLP_DIM], W0[MLP_DIM:], b0.reshape(1, MLP_DIM),
                  W1, b1.reshape(1, FACTOR), Wp[:FACTOR].reshape(1, FACTOR),
                  Wp[FACTOR:].reshape(1, FACTOR), bp.reshape(1, 1))
